# baseline (speedup 1.0000x reference)
import jax
import jax.numpy as jnp
from jax import lax
from jax.experimental import pallas as pl
from jax.experimental.pallas import tpu as pltpu
from jax.experimental.pallas import tpu_sc as plsc

_DIM = 32
_B = 16384
_S = 50
_NIDX = _B * _S
_CH = 16384
_NCHUNK = 62
_TROWS = _NCHUNK * _CH

_NW = 32
_CHUNK = 512
_NBUF = 4


def _relayout_table(w_t):
    def body(x_ref, o_ref):
        x = x_ref[...]
        z = jnp.concatenate(
            [x[:, 0:4096], x[:, 4096:8192], x[:, 8192:12288], x[:, 12288:16384]],
            axis=0,
        )
        o_ref[...] = z.T

    return pl.pallas_call(
        body,
        grid=(_NCHUNK,),
        in_specs=[pl.BlockSpec((_DIM, _CH), lambda i: (0, i))],
        out_specs=pl.BlockSpec((_CH // 4, 128), lambda i: (i, 0)),
        out_shape=jax.ShapeDtypeStruct((_TROWS // 4, 128), jnp.float32),
    )(w_t)


def _sc_gather(table, flat_ids, part, nparts):
    num_idx = (_S // nparts) * _B
    pstart = part * num_idx
    per_worker = num_idx // _NW
    nchunks = per_worker // _CHUNK
    qgroup = _B // 4
    splanes = _S // nparts
    mesh = plsc.VectorSubcoreMesh(core_axis_name="c", subcore_axis_name="s")

    @pl.kernel(
        out_type=jax.ShapeDtypeStruct((num_idx // 4, 4, _DIM), table.dtype),
        mesh=mesh,
        scratch_types=[
            pltpu.VMEM((_NBUF, _CHUNK), jnp.int32),
            pltpu.VMEM((_NBUF, _CHUNK, _DIM), jnp.float32),
            pltpu.SemaphoreType.DMA((_NBUF,)),
            pltpu.SemaphoreType.DMA((_NBUF,)),
            pltpu.SemaphoreType.DMA((_NBUF,)),
        ],
        compiler_params=pltpu.CompilerParams(use_tc_tiling_on_sc=False),
    )
    def gather_kernel(table_hbm, idx_hbm, out_hbm, idx_v, rows_v, isem, gsem, osem):
        wid = lax.axis_index("s") * 2 + lax.axis_index("c")
        base = pstart + wid * per_worker

        def dst(off):
            s_idx = off // _B
            rem = off - s_idx * _B
            q = rem // qgroup
            r0 = rem - q * qgroup
            return out_hbm.at[pl.ds((s_idx - part * splanes) * qgroup + r0, _CHUNK), q]

        def idx_copy(i, b):
            pltpu.async_copy(
                idx_hbm.at[pl.ds(base + i * _CHUNK, _CHUNK)], idx_v.at[b], isem.at[b]
            )

        def body(i, b, guarded):
            pltpu.make_async_copy(
                idx_hbm.at[pl.ds(base + i * _CHUNK, _CHUNK)], idx_v.at[b], isem.at[b]
            ).wait()

            def wait_wb():
                pltpu.make_async_copy(rows_v.at[b], dst(base), osem.at[b]).wait()

            if guarded:
                pl.when(i >= _NBUF)(wait_wb)
            elif i >= _NBUF:
                wait_wb()

            pltpu.async_copy(table_hbm.at[idx_v.at[b]], rows_v.at[b], gsem.at[b])

            pb = (b - 1) % _NBUF
            j = i - 1

            def retire_prev():
                pltpu.make_async_copy(
                    table_hbm.at[idx_v.at[pb]], rows_v.at[pb], gsem.at[pb]
                ).wait()
                pltpu.async_copy(rows_v.at[pb], dst(base + j * _CHUNK), osem.at[pb])

            if guarded:
                pl.when(j >= 0)(retire_prev)
            elif j >= 0:
                retire_prev()

            nb = (b + 2) % _NBUF

            def prefetch():
                idx_copy(i + 2, nb)

            if guarded:
                pl.when(i + 2 < nchunks)(prefetch)
            elif i + 2 < nchunks:
                prefetch()

        idx_copy(0, 0)
        idx_copy(1, 1)

        nloop = (nchunks - 2) // _NBUF

        @pl.loop(0, nloop)
        def _(g):
            for b in range(_NBUF):
                body(g * _NBUF + b, b, guarded=True)

        for i in range(nloop * _NBUF, nchunks):
            body(i, i % _NBUF, guarded=False)

        lb = (nchunks - 1) % _NBUF
        pltpu.make_async_copy(
            table_hbm.at[idx_v.at[lb]], rows_v.at[lb], gsem.at[lb]
        ).wait()
        pltpu.async_copy(
            rows_v.at[lb], dst(base + (nchunks - 1) * _CHUNK), osem.at[lb]
        )
        for i in range(nchunks - _NBUF, nchunks):
            b = i % _NBUF
            pltpu.make_async_copy(rows_v.at[b], dst(base), osem.at[b]).wait()

    return gather_kernel(table, flat_ids)


def _emit_part(o2, g3p, part, nparts):
    splanes = _S // nparts

    def body(x_ref, *refs):
        o_ref = refs[-1]
        z = x_ref[0].T
        o_ref[0] = jnp.concatenate([z[0:32], z[32:64], z[64:96], z[96:128]], axis=1)

    in_specs = [pl.BlockSpec((1, _B // 4, 128), lambda s: (s, 0, 0))]
    operands = [g3p]
    aliases = {}
    if o2 is not None:
        in_specs.append(pl.BlockSpec((1, 8, 128), lambda s: (0, 0, 0)))
        operands.append(o2)
        aliases = {1: 0}

    return pl.pallas_call(
        body,
        grid=(splanes,),
        in_specs=in_specs,
        out_specs=pl.BlockSpec(
            (1, _DIM, _B), lambda s, part=part, splanes=splanes: (part * splanes + s, 0, 0)
        ),
        out_shape=jax.ShapeDtypeStruct((_S, _DIM, _B), jnp.float32),
        input_output_aliases=aliases,
    )(*operands)


_P = 10


def kernel(token_ids, weights):
    ids = token_ids.astype(jnp.int32).T.reshape(-1)
    u = ids & (_CH - 1)
    rho = (ids - u) + 4 * (u & (_CH // 4 - 1)) + (u >> 12)

    table = _relayout_table(weights.T).reshape(_TROWS, _DIM)
    o2 = None
    for p in range(_P):
        g = _sc_gather(table, rho, p, _P)
        o2 = _emit_part(o2, g.reshape(_S // _P, _B // 4, 128), p, _P)
    return jnp.transpose(o2, (2, 0, 1))


# speedup vs baseline: 1.0350x; 1.0350x over previous
import jax
import jax.numpy as jnp
from jax import lax
from jax.experimental import pallas as pl
from jax.experimental.pallas import tpu as pltpu
from jax.experimental.pallas import tpu_sc as plsc

_DIM = 32
_B = 16384
_S = 50
_NIDX = _B * _S
_CH = 16384
_NCHUNK = 62
_TROWS = _NCHUNK * _CH

_NW = 32
_CHUNK = 512
_NBUF = 6
_LAG = 2


def _relayout_table(w_t):
    def body(x_ref, o_ref):
        x = x_ref[...]
        z = jnp.concatenate(
            [x[:, 0:4096], x[:, 4096:8192], x[:, 8192:12288], x[:, 12288:16384]],
            axis=0,
        )
        o_ref[...] = z.T

    return pl.pallas_call(
        body,
        grid=(_NCHUNK,),
        in_specs=[pl.BlockSpec((_DIM, _CH), lambda i: (0, i))],
        out_specs=pl.BlockSpec((_CH // 4, 128), lambda i: (i, 0)),
        out_shape=jax.ShapeDtypeStruct((_TROWS // 4, 128), jnp.float32),
    )(w_t)


def _sc_gather(table, flat_ids, part, nparts):
    num_idx = (_S // nparts) * _B
    pstart = part * num_idx
    per_worker = num_idx // _NW
    nchunks = per_worker // _CHUNK
    qgroup = _B // 4
    splanes = _S // nparts
    mesh = plsc.VectorSubcoreMesh(core_axis_name="c", subcore_axis_name="s")

    @pl.kernel(
        out_type=jax.ShapeDtypeStruct((num_idx // 4, 4, _DIM), table.dtype),
        mesh=mesh,
        scratch_types=[
            pltpu.VMEM((_NBUF, _CHUNK), jnp.int32),
            pltpu.VMEM((_NBUF, _CHUNK, _DIM), jnp.float32),
            pltpu.SemaphoreType.DMA((_NBUF,)),
            pltpu.SemaphoreType.DMA((_NBUF,)),
            pltpu.SemaphoreType.DMA((_NBUF,)),
        ],
        compiler_params=pltpu.CompilerParams(use_tc_tiling_on_sc=False),
    )
    def gather_kernel(table_hbm, idx_hbm, out_hbm, idx_v, rows_v, isem, gsem, osem):
        wid = lax.axis_index("s") * 2 + lax.axis_index("c")
        base = pstart + wid * per_worker

        def dst(off):
            s_idx = off // _B
            rem = off - s_idx * _B
            q = rem // qgroup
            r0 = rem - q * qgroup
            return out_hbm.at[pl.ds((s_idx - part * splanes) * qgroup + r0, _CHUNK), q]

        def idx_copy(i, b):
            pltpu.async_copy(
                idx_hbm.at[pl.ds(base + i * _CHUNK, _CHUNK)], idx_v.at[b], isem.at[b]
            )

        def body(i, b, guarded):
            pltpu.make_async_copy(
                idx_hbm.at[pl.ds(base + i * _CHUNK, _CHUNK)], idx_v.at[b], isem.at[b]
            ).wait()

            def wait_wb():
                pltpu.make_async_copy(rows_v.at[b], dst(base), osem.at[b]).wait()

            if guarded:
                pl.when(i >= _NBUF)(wait_wb)
            elif i >= _NBUF:
                wait_wb()

            pltpu.async_copy(table_hbm.at[idx_v.at[b]], rows_v.at[b], gsem.at[b])

            pb = (b - _LAG) % _NBUF
            j = i - _LAG

            def retire_prev():
                pltpu.make_async_copy(
                    table_hbm.at[idx_v.at[pb]], rows_v.at[pb], gsem.at[pb]
                ).wait()
                pltpu.async_copy(rows_v.at[pb], dst(base + j * _CHUNK), osem.at[pb])

            if guarded:
                pl.when(j >= 0)(retire_prev)
            elif j >= 0:
                retire_prev()

            nb = (b + _LAG + 2) % _NBUF

            def prefetch():
                idx_copy(i + _LAG + 2, nb)

            if guarded:
                pl.when(i + _LAG + 2 < nchunks)(prefetch)
            elif i + _LAG + 2 < nchunks:
                prefetch()

        for k in range(min(_LAG + 2, nchunks)):
            idx_copy(k, k)

        nloop = (nchunks - _LAG) // _NBUF

        @pl.loop(0, nloop)
        def _(g):
            for b in range(_NBUF):
                body(g * _NBUF + b, b, guarded=True)

        for i in range(nloop * _NBUF, nchunks):
            body(i, i % _NBUF, guarded=False)

        for k in range(nchunks - _LAG, nchunks):
            kb = k % _NBUF
            pltpu.make_async_copy(
                table_hbm.at[idx_v.at[kb]], rows_v.at[kb], gsem.at[kb]
            ).wait()
            pltpu.async_copy(rows_v.at[kb], dst(base + k * _CHUNK), osem.at[kb])
        for i in range(nchunks - _NBUF, nchunks):
            b = i % _NBUF
            pltpu.make_async_copy(rows_v.at[b], dst(base), osem.at[b]).wait()

    return gather_kernel(table, flat_ids)


def _emit_part(o2, g3p, part, nparts):
    splanes = _S // nparts

    def body(x_ref, *refs):
        o_ref = refs[-1]
        z = x_ref[0].T
        o_ref[0] = jnp.concatenate([z[0:32], z[32:64], z[64:96], z[96:128]], axis=1)

    in_specs = [pl.BlockSpec((1, _B // 4, 128), lambda s: (s, 0, 0))]
    operands = [g3p]
    aliases = {}
    if o2 is not None:
        in_specs.append(pl.BlockSpec((1, 8, 128), lambda s: (0, 0, 0)))
        operands.append(o2)
        aliases = {1: 0}

    return pl.pallas_call(
        body,
        grid=(splanes,),
        in_specs=in_specs,
        out_specs=pl.BlockSpec(
            (1, _DIM, _B), lambda s, part=part, splanes=splanes: (part * splanes + s, 0, 0)
        ),
        out_shape=jax.ShapeDtypeStruct((_S, _DIM, _B), jnp.float32),
        input_output_aliases=aliases,
    )(*operands)


_P = 5


def kernel(token_ids, weights):
    ids = token_ids.astype(jnp.int32).T.reshape(-1)
    u = ids & (_CH - 1)
    rho = (ids - u) + 4 * (u & (_CH // 4 - 1)) + (u >> 12)

    table = _relayout_table(weights.T).reshape(_TROWS, _DIM)
    o2 = None
    for p in range(_P):
        g = _sc_gather(table, rho, p, _P)
        o2 = _emit_part(o2, g.reshape(_S // _P, _B // 4, 128), p, _P)
    return jnp.transpose(o2, (2, 0, 1))


# speedup vs baseline: 1.0367x; 1.0017x over previous
import jax
import jax.numpy as jnp
from jax import lax
from jax.experimental import pallas as pl
from jax.experimental.pallas import tpu as pltpu
from jax.experimental.pallas import tpu_sc as plsc

_DIM = 32
_B = 16384
_S = 50
_NIDX = _B * _S
_CH = 16384
_NCHUNK = 62
_TROWS = _NCHUNK * _CH

_NW = 32
_CHUNK = 512
_NBUF = 4
_LAG = 1


def _relayout_table(w_t):
    def body(x_ref, o_ref):
        x = x_ref[...]
        z = jnp.concatenate(
            [x[:, 0:4096], x[:, 4096:8192], x[:, 8192:12288], x[:, 12288:16384]],
            axis=0,
        )
        o_ref[...] = z.T

    return pl.pallas_call(
        body,
        grid=(_NCHUNK,),
        in_specs=[pl.BlockSpec((_DIM, _CH), lambda i: (0, i))],
        out_specs=pl.BlockSpec((_CH // 4, 128), lambda i: (i, 0)),
        out_shape=jax.ShapeDtypeStruct((_TROWS // 4, 128), jnp.float32),
    )(w_t)


def _sc_gather(table, flat_ids, part, nparts):
    num_idx = (_S // nparts) * _B
    pstart = part * num_idx
    per_worker = num_idx // _NW
    nchunks = per_worker // _CHUNK
    qgroup = _B // 4
    splanes = _S // nparts
    mesh = plsc.VectorSubcoreMesh(core_axis_name="c", subcore_axis_name="s")

    @pl.kernel(
        out_type=jax.ShapeDtypeStruct((num_idx // 4, 4, _DIM), table.dtype),
        mesh=mesh,
        scratch_types=[
            pltpu.VMEM((_NBUF, _CHUNK), jnp.int32),
            pltpu.VMEM((_NBUF, _CHUNK, _DIM), jnp.float32),
            pltpu.SemaphoreType.DMA((_NBUF,)),
            pltpu.SemaphoreType.DMA((_NBUF,)),
            pltpu.SemaphoreType.DMA((_NBUF,)),
        ],
        compiler_params=pltpu.CompilerParams(use_tc_tiling_on_sc=False),
    )
    def gather_kernel(table_hbm, idx_hbm, out_hbm, idx_v, rows_v, isem, gsem, osem):
        wid = lax.axis_index("s") * 2 + lax.axis_index("c")
        base = pstart + wid * per_worker

        def dst(off):
            s_idx = off // _B
            rem = off - s_idx * _B
            q = rem // qgroup
            r0 = rem - q * qgroup
            return out_hbm.at[pl.ds((s_idx - part * splanes) * qgroup + r0, _CHUNK), q]

        def idx_copy(i, b):
            pltpu.async_copy(
                idx_hbm.at[pl.ds(base + i * _CHUNK, _CHUNK)], idx_v.at[b], isem.at[b]
            )

        def body(i, b, guarded):
            pltpu.make_async_copy(
                idx_hbm.at[pl.ds(base + i * _CHUNK, _CHUNK)], idx_v.at[b], isem.at[b]
            ).wait()

            def wait_wb():
                pltpu.make_async_copy(rows_v.at[b], dst(base), osem.at[b]).wait()

            if guarded:
                pl.when(i >= _NBUF)(wait_wb)
            elif i >= _NBUF:
                wait_wb()

            pltpu.async_copy(table_hbm.at[idx_v.at[b]], rows_v.at[b], gsem.at[b])

            pb = (b - _LAG) % _NBUF
            j = i - _LAG

            def retire_prev():
                pltpu.make_async_copy(
                    table_hbm.at[idx_v.at[pb]], rows_v.at[pb], gsem.at[pb]
                ).wait()
                pltpu.async_copy(rows_v.at[pb], dst(base + j * _CHUNK), osem.at[pb])

            if guarded:
                pl.when(j >= 0)(retire_prev)
            elif j >= 0:
                retire_prev()

            nb = (b + _LAG + 2) % _NBUF

            def prefetch():
                idx_copy(i + _LAG + 2, nb)

            if guarded:
                pl.when(i + _LAG + 2 < nchunks)(prefetch)
            elif i + _LAG + 2 < nchunks:
                prefetch()

        for k in range(min(_LAG + 2, nchunks)):
            idx_copy(k, k)

        nloop = (nchunks - _LAG) // _NBUF

        @pl.loop(0, nloop)
        def _(g):
            for b in range(_NBUF):
                body(g * _NBUF + b, b, guarded=True)

        for i in range(nloop * _NBUF, nchunks):
            body(i, i % _NBUF, guarded=False)

        for k in range(nchunks - _LAG, nchunks):
            kb = k % _NBUF
            pltpu.make_async_copy(
                table_hbm.at[idx_v.at[kb]], rows_v.at[kb], gsem.at[kb]
            ).wait()
            pltpu.async_copy(rows_v.at[kb], dst(base + k * _CHUNK), osem.at[kb])
        for i in range(nchunks - _NBUF, nchunks):
            b = i % _NBUF
            pltpu.make_async_copy(rows_v.at[b], dst(base), osem.at[b]).wait()

    return gather_kernel(table, flat_ids)


def _emit_part(o2, g3p, part, nparts):
    splanes = _S // nparts

    def body(x_ref, *refs):
        o_ref = refs[-1]
        z = x_ref[0].T
        o_ref[0] = jnp.concatenate([z[0:32], z[32:64], z[64:96], z[96:128]], axis=1)

    in_specs = [pl.BlockSpec((1, _B // 4, 128), lambda s: (s, 0, 0))]
    operands = [g3p]
    aliases = {}
    if o2 is not None:
        in_specs.append(pl.BlockSpec((1, 8, 128), lambda s: (0, 0, 0)))
        operands.append(o2)
        aliases = {1: 0}

    return pl.pallas_call(
        body,
        grid=(splanes,),
        in_specs=in_specs,
        out_specs=pl.BlockSpec(
            (1, _DIM, _B), lambda s, part=part, splanes=splanes: (part * splanes + s, 0, 0)
        ),
        out_shape=jax.ShapeDtypeStruct((_S, _DIM, _B), jnp.float32),
        input_output_aliases=aliases,
    )(*operands)


_P = 5


def kernel(token_ids, weights):
    ids = token_ids.astype(jnp.int32).T.reshape(-1)
    u = ids & (_CH - 1)
    rho = (ids - u) + 4 * (u & (_CH // 4 - 1)) + (u >> 12)

    table = _relayout_table(weights.T).reshape(_TROWS, _DIM)
    o2 = None
    for p in range(_P):
        g = _sc_gather(table, rho, p, _P)
        o2 = _emit_part(o2, g.reshape(_S // _P, _B // 4, 128), p, _P)
    return jnp.transpose(o2, (2, 0, 1))
